# baseline (speedup 1.0000x reference)
import numpy as np

import jax
import jax.numpy as jnp
from jax import lax
from jax.experimental import pallas as pl
from jax.experimental.pallas import tpu as pltpu
from jax.experimental.pallas import tpu_sc as plsc

H, W = 512, 512
HW = H * W
NB = 4
NSLOT = 8
SLOT_PIX = HW // NSLOT
CHUNK = 4096
NCHUNK = HW // CHUNK
GROUPS = CHUNK // 16
DEDUP = 2048
INIT_KEY = 3.0e38
BIG_IDX = 2 ** 30

_K_MAT = np.array([[512.0, 0.0, 256.0, 0.0],
                   [0.0, 512.0, 256.0, 0.0],
                   [0.0, 0.0, 1.0, 0.0],
                   [0.0, 0.0, 0.0, 1.0]], dtype=np.float32)
_INV_K = np.array([[1.0 / 512.0, 0.0, -0.5, 0.0],
                   [0.0, 1.0 / 512.0, -0.5, 0.0],
                   [0.0, 0.0, 1.0, 0.0],
                   [0.0, 0.0, 0.0, 1.0]], dtype=np.float32)


def _pix_coords_k():
    ys, xs = jnp.meshgrid(jnp.arange(H, dtype=jnp.float32),
                          jnp.arange(W, dtype=jnp.float32), indexing='ij')
    ones = jnp.ones((H * W,), dtype=jnp.float32)
    return jnp.stack([xs.reshape(-1), ys.reshape(-1), ones], axis=0)


def _project_points(depth, T):
    b = depth.shape[0]
    pix = _pix_coords_k()
    cam = jnp.einsum('ij,jk->ik', jnp.asarray(_INV_K)[:3, :3], pix)
    cam = depth.reshape(b, 1, H * W) * cam[None]
    ones = jnp.ones((b, 1, H * W), dtype=depth.dtype)
    pts3d = jnp.concatenate([cam, ones], axis=1)
    pts3d_nv = jnp.einsum('bij,bjk->bik', T, pts3d)
    P = jnp.einsum('ij,bjk->bik', jnp.asarray(_K_MAT)[:3, :], pts3d_nv)
    pix_uv = P[:, :2] / (P[:, 2:3] + 1e-7)
    return pix_uv[:, 0], pix_uv[:, 1], pts3d_nv[:, 2]


def _prep_body(u_ref, v_ref, d_ref, r_ref, g_ref, b_ref, uvid_ref, rgbw_ref):
    u = u_ref[0]
    v = v_ref[0]
    d = d_ref[0]
    valid = (d > 0.0) & (d < 1000000.0)
    uc = jnp.clip(jnp.round(u), 0.0, W - 1.0)
    vc = jnp.clip(jnp.round(v), 0.0, H - 1.0)
    pid = (vc * W + uc).astype(jnp.int32)
    uvid_ref[0] = jnp.where(valid, pid, HW)
    ri = jnp.clip(jnp.round(r_ref[0] * 2047.0), 0.0, 2047.0).astype(jnp.int32)
    gi = jnp.clip(jnp.round(g_ref[0] * 2047.0), 0.0, 2047.0).astype(jnp.int32)
    bi = jnp.clip(jnp.round(b_ref[0] * 1023.0), 0.0, 1023.0).astype(jnp.int32)
    rgbw_ref[0] = (ri << 21) | (gi << 10) | bi


def _prep(u, v, d, r, g, b):
    spec = pl.BlockSpec((1, H, W), lambda i: (i, 0, 0))
    return pl.pallas_call(
        _prep_body,
        grid=(NB,),
        in_specs=[spec] * 6,
        out_specs=[spec, spec],
        out_shape=[jax.ShapeDtypeStruct((NB, H, W), jnp.int32),
                   jax.ShapeDtypeStruct((NB, H, W), jnp.int32)],
    )(u, v, d, r, g, b)


def _zbuffer_body(uvid_hbm, key_hbm, rgbw_hbm, keyout, rgbout,
                  keybuf, idxbuf, rgbbuf,
                  uv0, ke0, rw0, uv1, ke1, rw1, sem0, sem1):
    wid = lax.axis_index("s") * 2 + lax.axis_index("c")
    bidx = wid // NSLOT
    slot = wid % NSLOT
    base_pt = bidx * HW
    base_px = bidx * HW + slot * SLOT_PIX
    lanes = lax.iota(jnp.int32, 16)

    def init_body(j, _):
        keybuf[pl.ds(j * 16, 16)] = jnp.full((16,), INIT_KEY, jnp.float32)
        idxbuf[pl.ds(j * 16, 16)] = jnp.full((16,), BIG_IDX, jnp.int32)
        return 0

    lax.fori_loop(0, SLOT_PIX // 16, init_body, 0)

    bufs = ((uv0, ke0, rw0, sem0), (uv1, ke1, rw1, sem1))

    def issue(s, ci):
        off = base_pt + ci * CHUNK
        uv, ke, rw, sem = bufs[s]
        pltpu.async_copy(uvid_hbm.at[pl.ds(off, CHUNK)], uv, sem)
        pltpu.async_copy(key_hbm.at[pl.ds(off, CHUNK)], ke, sem)
        pltpu.async_copy(rgbw_hbm.at[pl.ds(off, CHUNK)], rw, sem)

    def wait(s, ci):
        off = base_pt + ci * CHUNK
        uv, ke, rw, sem = bufs[s]
        pltpu.make_async_copy(uvid_hbm.at[pl.ds(off, CHUNK)], uv, sem).wait()
        pltpu.make_async_copy(key_hbm.at[pl.ds(off, CHUNK)], ke, sem).wait()
        pltpu.make_async_copy(rgbw_hbm.at[pl.ds(off, CHUNK)], rw, sem).wait()

    big_vec = jnp.full((16,), BIG_IDX, jnp.int32)
    GUNROLL = 8

    def process(s, ci):
        uv, ke, rw, _ = bufs[s]
        ibase = ci * CHUNK

        def supergroup(sg, sticky):
            g0 = sg * GUNROLL
            J = range(GUNROLL)
            pid = [uv[pl.ds((g0 + j) * 16, 16)] for j in J]
            k = [ke[pl.ds((g0 + j) * 16, 16)] for j in J]
            w = [rw[pl.ds((g0 + j) * 16, 16)] for j in J]
            i_vec = [(ibase + (g0 + j) * 16) + lanes for j in J]
            own = [lax.shift_right_logical(pid[j], 15) == slot for j in J]
            lpid = [pid[j] & (SLOT_PIX - 1) for j in J]
            curk = [plsc.load_gather(keybuf, [lpid[j]]) for j in J]
            act = [own[j] & (k[j] < curk[j]) for j in J]
            for j in J:
                plsc.store_scatter(keybuf, [lpid[j]], k[j], mask=act[j])
            for j in J:
                plsc.store_scatter(idxbuf, [lpid[j]], big_vec, mask=act[j])
            curk2 = [plsc.load_gather(keybuf, [lpid[j]]) for j in J]
            still_a = [act[j] & (k[j] < curk2[j]) for j in J]
            won = [own[j] & (k[j] == curk2[j]) for j in J]
            curi = [plsc.load_gather(idxbuf, [lpid[j]]) for j in J]
            bet = [won[j] & (i_vec[j] < curi[j]) for j in J]
            for j in J:
                plsc.store_scatter(idxbuf, [lpid[j]], i_vec[j], mask=bet[j])
            curi2 = [plsc.load_gather(idxbuf, [lpid[j]]) for j in J]
            still_b = [bet[j] & (i_vec[j] < curi2[j]) for j in J]
            rgbm = [won[j] & (i_vec[j] == curi2[j]) for j in J]
            for j in J:
                plsc.store_scatter(rgbbuf, [lpid[j]], w[j], mask=rgbm[j])
            for j in J:
                sticky = sticky | still_a[j] | still_b[j]
            return sticky

        def pass_cond(st):
            return st[1] > 0

        def pass_body(st):
            sticky = lax.fori_loop(0, GROUPS // GUNROLL, supergroup,
                                   jnp.zeros((16,), jnp.bool_))
            return (0, jnp.max(sticky.astype(jnp.int32)))

        lax.while_loop(pass_cond, pass_body, (0, jnp.int32(1)))

    issue(0, 0)
    issue(1, 1)

    def pair_body(cp, _):
        for s in range(2):
            ci = cp * 2 + s
            wait(s, ci)
            process(s, ci)

            @pl.when(ci + 2 < NCHUNK)
            def _():
                issue(s, ci + 2)
        return 0

    lax.fori_loop(0, NCHUNK // 2, pair_body, 0)

    pltpu.sync_copy(keybuf, keyout.at[pl.ds(base_px, SLOT_PIX)])
    pltpu.sync_copy(rgbbuf, rgbout.at[pl.ds(base_px, SLOT_PIX)])


def _zbuffer(uvid, key, rgbw):
    mesh = plsc.VectorSubcoreMesh(core_axis_name="c", subcore_axis_name="s")
    fn = pl.kernel(
        _zbuffer_body,
        out_type=(jax.ShapeDtypeStruct((NB * HW,), jnp.float32),
                  jax.ShapeDtypeStruct((NB * HW,), jnp.int32)),
        mesh=mesh,
        scratch_types=[
            pltpu.VMEM((SLOT_PIX,), jnp.float32),
            pltpu.VMEM((SLOT_PIX,), jnp.int32),
            pltpu.VMEM((SLOT_PIX,), jnp.int32),
            pltpu.VMEM((CHUNK,), jnp.int32),
            pltpu.VMEM((CHUNK,), jnp.float32),
            pltpu.VMEM((CHUNK,), jnp.int32),
            pltpu.VMEM((CHUNK,), jnp.int32),
            pltpu.VMEM((CHUNK,), jnp.float32),
            pltpu.VMEM((CHUNK,), jnp.int32),
            pltpu.SemaphoreType.DMA,
            pltpu.SemaphoreType.DMA,
        ],
        compiler_params=pltpu.CompilerParams(needs_layout_passes=False),
    )
    return fn(uvid, key, rgbw)


def _shift01(x):
    return jnp.concatenate([x[:1], x[:-1]], axis=0)


def _shift10(x):
    return jnp.concatenate([x[:, :1], x[:, :-1]], axis=1)


def _shift11(x):
    inner = jnp.concatenate([x[1:, :1], x[:-1, :-1]], axis=1)
    return jnp.concatenate([x[:1], inner], axis=0)


def _epi_body(key_ref, rgb_ref, img_ref, dep_ref, mask_ref):
    key = key_ref[0]
    w = rgb_ref[0]
    has = key < 1.0e7
    dep = jnp.where(has, key, 1.0e8)
    r = jnp.where(has, ((w >> 21) & 2047).astype(jnp.float32) * (1.0 / 2047.0), 0.0)
    g = jnp.where(has, ((w >> 10) & 2047).astype(jnp.float32) * (1.0 / 2047.0), 0.0)
    b = jnp.where(has, (w & 1023).astype(jnp.float32) * (1.0 / 1023.0), 0.0)

    q = W // 4
    d0 = dep[:, q:3 * q]
    d1 = _shift01(d0)
    d2 = _shift10(d0)
    d3 = _shift11(d0)
    m1 = d1 < d0
    bd = jnp.where(m1, d1, d0)
    m2 = d2 < bd
    bd = jnp.where(m2, d2, bd)
    m3 = d3 < bd
    bd = jnp.where(m3, d3, bd)

    def merge_chan(c):
        c0 = c[:, q:3 * q]
        bc = jnp.where(m1, _shift01(c0), c0)
        bc = jnp.where(m2, _shift10(c0), bc)
        bc = jnp.where(m3, _shift11(c0), bc)
        return jnp.concatenate([c[:, :q], bc, c[:, 3 * q:]], axis=1)

    img_ref[0, 0] = merge_chan(r)
    img_ref[0, 1] = merge_chan(g)
    img_ref[0, 2] = merge_chan(b)
    dep_out = jnp.concatenate([dep[:, :q], bd, dep[:, 3 * q:]], axis=1)
    dep_ref[0, 0] = dep_out
    mask_ref[0, 0] = (dep_out > 1000000.0).astype(jnp.float32)


def _epilogue(keyimg, rgbimg):
    spec2 = pl.BlockSpec((1, H, W), lambda i: (i, 0, 0))
    return pl.pallas_call(
        _epi_body,
        grid=(NB,),
        in_specs=[spec2, spec2],
        out_specs=[pl.BlockSpec((1, 3, H, W), lambda i: (i, 0, 0, 0)),
                   pl.BlockSpec((1, 1, H, W), lambda i: (i, 0, 0, 0)),
                   pl.BlockSpec((1, 1, H, W), lambda i: (i, 0, 0, 0))],
        out_shape=[jax.ShapeDtypeStruct((NB, 3, H, W), jnp.float32),
                   jax.ShapeDtypeStruct((NB, 1, H, W), jnp.float32),
                   jax.ShapeDtypeStruct((NB, 1, H, W), jnp.float32)],
    )(keyimg, rgbimg)


def kernel(img, depth, T):
    u, v, d = _project_points(depth, T)
    u = u.reshape(NB, H, W)
    v = v.reshape(NB, H, W)
    d3 = d.reshape(NB, H, W)
    uvid, rgbw = _prep(u, v, d3, img[:, 0], img[:, 1], img[:, 2])
    keyout, rgbout = _zbuffer(uvid.reshape(NB * HW),
                              d.reshape(NB * HW),
                              rgbw.reshape(NB * HW))
    nv_img, nv_depth, nv_mask = _epilogue(keyout.reshape(NB, H, W),
                                          rgbout.reshape(NB, H, W))
    return nv_img, nv_depth, nv_mask


# speedup vs baseline: 1.4932x; 1.4932x over previous
import numpy as np

import jax
import jax.numpy as jnp
from jax import lax
from jax.experimental import pallas as pl
from jax.experimental.pallas import tpu as pltpu
from jax.experimental.pallas import tpu_sc as plsc

H, W = 512, 512
HW = H * W
NB = 4
NSLOT = 8
SLOT_PIX = HW // NSLOT
CHUNK = 2048
NCHUNK = HW // CHUNK
GROUPS = CHUNK // 16
DEDUP = 2048
INIT_KEY = 3.0e38
BIG_IDX = 2 ** 30

_K_MAT = np.array([[512.0, 0.0, 256.0, 0.0],
                   [0.0, 512.0, 256.0, 0.0],
                   [0.0, 0.0, 1.0, 0.0],
                   [0.0, 0.0, 0.0, 1.0]], dtype=np.float32)
_INV_K = np.array([[1.0 / 512.0, 0.0, -0.5, 0.0],
                   [0.0, 1.0 / 512.0, -0.5, 0.0],
                   [0.0, 0.0, 1.0, 0.0],
                   [0.0, 0.0, 0.0, 1.0]], dtype=np.float32)


def _pix_coords_k():
    ys, xs = jnp.meshgrid(jnp.arange(H, dtype=jnp.float32),
                          jnp.arange(W, dtype=jnp.float32), indexing='ij')
    ones = jnp.ones((H * W,), dtype=jnp.float32)
    return jnp.stack([xs.reshape(-1), ys.reshape(-1), ones], axis=0)


def _project_points(depth, T):
    b = depth.shape[0]
    pix = _pix_coords_k()
    cam = jnp.einsum('ij,jk->ik', jnp.asarray(_INV_K)[:3, :3], pix)
    cam = depth.reshape(b, 1, H * W) * cam[None]
    ones = jnp.ones((b, 1, H * W), dtype=depth.dtype)
    pts3d = jnp.concatenate([cam, ones], axis=1)
    pts3d_nv = jnp.einsum('bij,bjk->bik', T, pts3d)
    P = jnp.einsum('ij,bjk->bik', jnp.asarray(_K_MAT)[:3, :], pts3d_nv)
    pix_uv = P[:, :2] / (P[:, 2:3] + 1e-7)
    return pix_uv[:, 0], pix_uv[:, 1], pts3d_nv[:, 2]


def _prep_body(u_ref, v_ref, d_ref, r_ref, g_ref, b_ref, uvid_ref, rgbw_ref):
    u = u_ref[0]
    v = v_ref[0]
    d = d_ref[0]
    valid = (d > 0.0) & (d < 1000000.0)
    uc = jnp.clip(jnp.round(u), 0.0, W - 1.0)
    vc = jnp.clip(jnp.round(v), 0.0, H - 1.0)
    pid = (vc * W + uc).astype(jnp.int32)
    uvid_ref[0] = jnp.where(valid, pid, HW)
    ri = jnp.clip(jnp.round(r_ref[0] * 2047.0), 0.0, 2047.0).astype(jnp.int32)
    gi = jnp.clip(jnp.round(g_ref[0] * 2047.0), 0.0, 2047.0).astype(jnp.int32)
    bi = jnp.clip(jnp.round(b_ref[0] * 1023.0), 0.0, 1023.0).astype(jnp.int32)
    rgbw_ref[0] = (ri << 21) | (gi << 10) | bi


def _prep(u, v, d, r, g, b):
    spec = pl.BlockSpec((1, H, W), lambda i: (i, 0, 0))
    return pl.pallas_call(
        _prep_body,
        grid=(NB,),
        in_specs=[spec] * 6,
        out_specs=[spec, spec],
        out_shape=[jax.ShapeDtypeStruct((NB, H, W), jnp.int32),
                   jax.ShapeDtypeStruct((NB, H, W), jnp.int32)],
    )(u, v, d, r, g, b)


def _zbuffer_body(uvid_hbm, key_hbm, rgbw_hbm, keyout, rgbout,
                  keybuf, idxbuf, rgbbuf,
                  uv0, ke0, rw0, uv1, ke1, rw1, sem0, sem1):
    wid = lax.axis_index("s") * 2 + lax.axis_index("c")
    bidx = wid // NSLOT
    slot = wid % NSLOT
    base_pt = bidx * HW
    base_px = bidx * HW + slot * SLOT_PIX
    lanes = lax.iota(jnp.int32, 16)

    def init_body(j, _):
        keybuf[pl.ds(j * 16, 16)] = jnp.full((16,), INIT_KEY, jnp.float32)
        idxbuf[pl.ds(j * 16, 16)] = jnp.full((16,), BIG_IDX, jnp.int32)
        return 0

    lax.fori_loop(0, SLOT_PIX // 16, init_body, 0)

    bufs = ((uv0, ke0, rw0, sem0), (uv1, ke1, rw1, sem1))

    def issue(s, ci):
        off = base_pt + ci * CHUNK
        uv, ke, rw, sem = bufs[s]
        pltpu.async_copy(uvid_hbm.at[pl.ds(off, CHUNK)], uv, sem)
        pltpu.async_copy(key_hbm.at[pl.ds(off, CHUNK)], ke, sem)
        pltpu.async_copy(rgbw_hbm.at[pl.ds(off, CHUNK)], rw, sem)

    def wait(s, ci):
        off = base_pt + ci * CHUNK
        uv, ke, rw, sem = bufs[s]
        pltpu.make_async_copy(uvid_hbm.at[pl.ds(off, CHUNK)], uv, sem).wait()
        pltpu.make_async_copy(key_hbm.at[pl.ds(off, CHUNK)], ke, sem).wait()
        pltpu.make_async_copy(rgbw_hbm.at[pl.ds(off, CHUNK)], rw, sem).wait()

    big_vec = jnp.full((16,), BIG_IDX, jnp.int32)
    GUNROLL = 8

    def process(s, ci):
        uv, ke, rw, _ = bufs[s]
        ibase = ci * CHUNK

        def supergroup(sg, sticky):
            g0 = sg * GUNROLL
            J = range(GUNROLL)
            pid = [uv[pl.ds((g0 + j) * 16, 16)] for j in J]
            k = [ke[pl.ds((g0 + j) * 16, 16)] for j in J]
            w = [rw[pl.ds((g0 + j) * 16, 16)] for j in J]
            i_vec = [(ibase + (g0 + j) * 16) + lanes for j in J]
            own = [lax.shift_right_logical(pid[j], 15) == slot for j in J]
            lpid = [pid[j] & (SLOT_PIX - 1) for j in J]
            curk = [plsc.load_gather(keybuf, [lpid[j]], mask=own[j]) for j in J]
            act = [own[j] & (k[j] < curk[j]) for j in J]
            for j in J:
                plsc.store_scatter(keybuf, [lpid[j]], k[j], mask=act[j])
            for j in J:
                plsc.store_scatter(idxbuf, [lpid[j]], big_vec, mask=act[j])
            curk2 = [plsc.load_gather(keybuf, [lpid[j]], mask=own[j]) for j in J]
            still_a = [act[j] & (k[j] < curk2[j]) for j in J]
            won = [own[j] & (k[j] == curk2[j]) for j in J]
            curi = [plsc.load_gather(idxbuf, [lpid[j]], mask=won[j]) for j in J]
            bet = [won[j] & (i_vec[j] < curi[j]) for j in J]
            for j in J:
                plsc.store_scatter(idxbuf, [lpid[j]], i_vec[j], mask=bet[j])
            curi2 = [plsc.load_gather(idxbuf, [lpid[j]], mask=won[j]) for j in J]
            still_b = [bet[j] & (i_vec[j] < curi2[j]) for j in J]
            rgbm = [won[j] & (i_vec[j] == curi2[j]) for j in J]
            for j in J:
                plsc.store_scatter(rgbbuf, [lpid[j]], w[j], mask=rgbm[j])
            for j in J:
                sticky = sticky | still_a[j] | still_b[j]
            return sticky

        def pass_cond(st):
            return st[1] > 0

        def pass_body(st):
            sticky = lax.fori_loop(0, GROUPS // GUNROLL, supergroup,
                                   jnp.zeros((16,), jnp.bool_))
            return (0, jnp.max(sticky.astype(jnp.int32)))

        lax.while_loop(pass_cond, pass_body, (0, jnp.int32(1)))

    issue(0, 0)
    issue(1, 1)

    def pair_body(cp, _):
        for s in range(2):
            ci = cp * 2 + s
            wait(s, ci)
            process(s, ci)

            @pl.when(ci + 2 < NCHUNK)
            def _():
                issue(s, ci + 2)
        return 0

    lax.fori_loop(0, NCHUNK // 2, pair_body, 0)

    pltpu.sync_copy(keybuf, keyout.at[pl.ds(base_px, SLOT_PIX)])
    pltpu.sync_copy(rgbbuf, rgbout.at[pl.ds(base_px, SLOT_PIX)])


def _zbuffer(uvid, key, rgbw):
    mesh = plsc.VectorSubcoreMesh(core_axis_name="c", subcore_axis_name="s")
    fn = pl.kernel(
        _zbuffer_body,
        out_type=(jax.ShapeDtypeStruct((NB * HW,), jnp.float32),
                  jax.ShapeDtypeStruct((NB * HW,), jnp.int32)),
        mesh=mesh,
        scratch_types=[
            pltpu.VMEM((SLOT_PIX,), jnp.float32),
            pltpu.VMEM((SLOT_PIX,), jnp.int32),
            pltpu.VMEM((SLOT_PIX,), jnp.int32),
            pltpu.VMEM((CHUNK,), jnp.int32),
            pltpu.VMEM((CHUNK,), jnp.float32),
            pltpu.VMEM((CHUNK,), jnp.int32),
            pltpu.VMEM((CHUNK,), jnp.int32),
            pltpu.VMEM((CHUNK,), jnp.float32),
            pltpu.VMEM((CHUNK,), jnp.int32),
            pltpu.SemaphoreType.DMA,
            pltpu.SemaphoreType.DMA,
        ],
        compiler_params=pltpu.CompilerParams(needs_layout_passes=False),
    )
    return fn(uvid, key, rgbw)


def _shift01(x):
    return jnp.concatenate([x[:1], x[:-1]], axis=0)


def _shift10(x):
    return jnp.concatenate([x[:, :1], x[:, :-1]], axis=1)


def _shift11(x):
    inner = jnp.concatenate([x[1:, :1], x[:-1, :-1]], axis=1)
    return jnp.concatenate([x[:1], inner], axis=0)


def _epi_body(key_ref, rgb_ref, img_ref, dep_ref, mask_ref):
    key = key_ref[0]
    w = rgb_ref[0]
    has = key < 1.0e7
    dep = jnp.where(has, key, 1.0e8)
    r = jnp.where(has, ((w >> 21) & 2047).astype(jnp.float32) * (1.0 / 2047.0), 0.0)
    g = jnp.where(has, ((w >> 10) & 2047).astype(jnp.float32) * (1.0 / 2047.0), 0.0)
    b = jnp.where(has, (w & 1023).astype(jnp.float32) * (1.0 / 1023.0), 0.0)

    q = W // 4
    d0 = dep[:, q:3 * q]
    d1 = _shift01(d0)
    d2 = _shift10(d0)
    d3 = _shift11(d0)
    m1 = d1 < d0
    bd = jnp.where(m1, d1, d0)
    m2 = d2 < bd
    bd = jnp.where(m2, d2, bd)
    m3 = d3 < bd
    bd = jnp.where(m3, d3, bd)

    def merge_chan(c):
        c0 = c[:, q:3 * q]
        bc = jnp.where(m1, _shift01(c0), c0)
        bc = jnp.where(m2, _shift10(c0), bc)
        bc = jnp.where(m3, _shift11(c0), bc)
        return jnp.concatenate([c[:, :q], bc, c[:, 3 * q:]], axis=1)

    img_ref[0, 0] = merge_chan(r)
    img_ref[0, 1] = merge_chan(g)
    img_ref[0, 2] = merge_chan(b)
    dep_out = jnp.concatenate([dep[:, :q], bd, dep[:, 3 * q:]], axis=1)
    dep_ref[0, 0] = dep_out
    mask_ref[0, 0] = (dep_out > 1000000.0).astype(jnp.float32)


def _epilogue(keyimg, rgbimg):
    spec2 = pl.BlockSpec((1, H, W), lambda i: (i, 0, 0))
    return pl.pallas_call(
        _epi_body,
        grid=(NB,),
        in_specs=[spec2, spec2],
        out_specs=[pl.BlockSpec((1, 3, H, W), lambda i: (i, 0, 0, 0)),
                   pl.BlockSpec((1, 1, H, W), lambda i: (i, 0, 0, 0)),
                   pl.BlockSpec((1, 1, H, W), lambda i: (i, 0, 0, 0))],
        out_shape=[jax.ShapeDtypeStruct((NB, 3, H, W), jnp.float32),
                   jax.ShapeDtypeStruct((NB, 1, H, W), jnp.float32),
                   jax.ShapeDtypeStruct((NB, 1, H, W), jnp.float32)],
    )(keyimg, rgbimg)


def kernel(img, depth, T):
    u, v, d = _project_points(depth, T)
    u = u.reshape(NB, H, W)
    v = v.reshape(NB, H, W)
    d3 = d.reshape(NB, H, W)
    uvid, rgbw = _prep(u, v, d3, img[:, 0], img[:, 1], img[:, 2])
    keyout, rgbout = _zbuffer(uvid.reshape(NB * HW),
                              d.reshape(NB * HW),
                              rgbw.reshape(NB * HW))
    nv_img, nv_depth, nv_mask = _epilogue(keyout.reshape(NB, H, W),
                                          rgbout.reshape(NB, H, W))
    return nv_img, nv_depth, nv_mask


# speedup vs baseline: 1.4934x; 1.0001x over previous
import numpy as np

import jax
import jax.numpy as jnp
from jax import lax
from jax.experimental import pallas as pl
from jax.experimental.pallas import tpu as pltpu
from jax.experimental.pallas import tpu_sc as plsc

H, W = 512, 512
HW = H * W
NB = 4
NSLOT = 8
SLOT_PIX = HW // NSLOT
CHUNK = 2048
NCHUNK = HW // CHUNK
GROUPS = CHUNK // 16
INIT_KEY = 3.0e38
BIG_IDX = 2 ** 30

_K_MAT = np.array([[512.0, 0.0, 256.0, 0.0],
                   [0.0, 512.0, 256.0, 0.0],
                   [0.0, 0.0, 1.0, 0.0],
                   [0.0, 0.0, 0.0, 1.0]], dtype=np.float32)
_INV_K = np.array([[1.0 / 512.0, 0.0, -0.5, 0.0],
                   [0.0, 1.0 / 512.0, -0.5, 0.0],
                   [0.0, 0.0, 1.0, 0.0],
                   [0.0, 0.0, 0.0, 1.0]], dtype=np.float32)


def _pix_coords_k():
    ys, xs = jnp.meshgrid(jnp.arange(H, dtype=jnp.float32),
                          jnp.arange(W, dtype=jnp.float32), indexing='ij')
    ones = jnp.ones((H * W,), dtype=jnp.float32)
    return jnp.stack([xs.reshape(-1), ys.reshape(-1), ones], axis=0)


def _project_points(depth, T):
    b = depth.shape[0]
    pix = _pix_coords_k()
    cam = jnp.einsum('ij,jk->ik', jnp.asarray(_INV_K)[:3, :3], pix)
    cam = depth.reshape(b, 1, H * W) * cam[None]
    ones = jnp.ones((b, 1, H * W), dtype=depth.dtype)
    pts3d = jnp.concatenate([cam, ones], axis=1)
    pts3d_nv = jnp.einsum('bij,bjk->bik', T, pts3d)
    P = jnp.einsum('ij,bjk->bik', jnp.asarray(_K_MAT)[:3, :], pts3d_nv)
    pix_uv = P[:, :2] / (P[:, 2:3] + 1e-7)
    return pix_uv[:, 0], pix_uv[:, 1], pts3d_nv[:, 2]


def _prep_body(u_ref, v_ref, d_ref, r_ref, g_ref, b_ref, uvid_ref, rgbw_ref):
    u = u_ref[0]
    v = v_ref[0]
    d = d_ref[0]
    valid = (d > 0.0) & (d < 1000000.0)
    uc = jnp.clip(jnp.round(u), 0.0, W - 1.0)
    vc = jnp.clip(jnp.round(v), 0.0, H - 1.0)
    pid = (vc * W + uc).astype(jnp.int32)
    uvid_ref[0] = jnp.where(valid, pid, HW)
    ri = jnp.clip(jnp.round(r_ref[0] * 2047.0), 0.0, 2047.0).astype(jnp.int32)
    gi = jnp.clip(jnp.round(g_ref[0] * 2047.0), 0.0, 2047.0).astype(jnp.int32)
    bi = jnp.clip(jnp.round(b_ref[0] * 1023.0), 0.0, 1023.0).astype(jnp.int32)
    rgbw_ref[0] = (ri << 21) | (gi << 10) | bi


def _prep(u, v, d, r, g, b):
    spec = pl.BlockSpec((1, H, W), lambda i: (i, 0, 0))
    return pl.pallas_call(
        _prep_body,
        grid=(NB,),
        in_specs=[spec] * 6,
        out_specs=[spec, spec],
        out_shape=[jax.ShapeDtypeStruct((NB, H, W), jnp.int32),
                   jax.ShapeDtypeStruct((NB, H, W), jnp.int32)],
    )(u, v, d, r, g, b)


def _zbuffer_body(uvid_hbm, key_hbm, rgbw_hbm, keyout, rgbout,
                  keybuf, idxbuf, rgbbuf,
                  uv0, ke0, rw0, uv1, ke1, rw1, sem0, sem1):
    wid = lax.axis_index("s") * 2 + lax.axis_index("c")
    bidx = wid // NSLOT
    slot = wid % NSLOT
    base_pt = bidx * HW
    base_px = bidx * HW + slot * SLOT_PIX
    lanes = lax.iota(jnp.int32, 16)

    def init_body(j, _):
        keybuf[pl.ds(j * 16, 16)] = jnp.full((16,), INIT_KEY, jnp.float32)
        idxbuf[pl.ds(j * 16, 16)] = jnp.full((16,), BIG_IDX, jnp.int32)
        return 0

    lax.fori_loop(0, SLOT_PIX // 16, init_body, 0)

    bufs = ((uv0, ke0, rw0, sem0), (uv1, ke1, rw1, sem1))

    def issue(s, ci):
        off = base_pt + ci * CHUNK
        uv, ke, rw, sem = bufs[s]
        pltpu.async_copy(uvid_hbm.at[pl.ds(off, CHUNK)], uv, sem)
        pltpu.async_copy(key_hbm.at[pl.ds(off, CHUNK)], ke, sem)
        pltpu.async_copy(rgbw_hbm.at[pl.ds(off, CHUNK)], rw, sem)

    def wait(s, ci):
        off = base_pt + ci * CHUNK
        uv, ke, rw, sem = bufs[s]
        pltpu.make_async_copy(uvid_hbm.at[pl.ds(off, CHUNK)], uv, sem).wait()
        pltpu.make_async_copy(key_hbm.at[pl.ds(off, CHUNK)], ke, sem).wait()
        pltpu.make_async_copy(rgbw_hbm.at[pl.ds(off, CHUNK)], rw, sem).wait()

    big_vec = jnp.full((16,), BIG_IDX, jnp.int32)
    GUNROLL = 8

    def process(s, ci):
        uv, ke, rw, _ = bufs[s]
        ibase = ci * CHUNK

        def supergroup(sg, sticky):
            g0 = sg * GUNROLL
            J = range(GUNROLL)
            pid = [uv[pl.ds((g0 + j) * 16, 16)] for j in J]
            k = [ke[pl.ds((g0 + j) * 16, 16)] for j in J]
            w = [rw[pl.ds((g0 + j) * 16, 16)] for j in J]
            i_vec = [(ibase + (g0 + j) * 16) + lanes for j in J]
            own = [lax.shift_right_logical(pid[j], 15) == slot for j in J]
            lpid = [pid[j] & (SLOT_PIX - 1) for j in J]
            curk = [plsc.load_gather(keybuf, [lpid[j]], mask=own[j]) for j in J]
            act = [own[j] & (k[j] < curk[j]) for j in J]
            for j in J:
                plsc.store_scatter(keybuf, [lpid[j]], k[j], mask=act[j])
            for j in J:
                plsc.store_scatter(idxbuf, [lpid[j]], big_vec, mask=act[j])
            curk2 = [plsc.load_gather(keybuf, [lpid[j]], mask=own[j]) for j in J]
            still_a = [act[j] & (k[j] < curk2[j]) for j in J]
            won = [own[j] & (k[j] == curk2[j]) for j in J]
            curi = [plsc.load_gather(idxbuf, [lpid[j]], mask=won[j]) for j in J]
            bet = [won[j] & (i_vec[j] < curi[j]) for j in J]
            for j in J:
                plsc.store_scatter(idxbuf, [lpid[j]], i_vec[j], mask=bet[j])
            curi2 = [plsc.load_gather(idxbuf, [lpid[j]], mask=won[j]) for j in J]
            still_b = [bet[j] & (i_vec[j] < curi2[j]) for j in J]
            rgbm = [won[j] & (i_vec[j] == curi2[j]) for j in J]
            for j in J:
                plsc.store_scatter(rgbbuf, [lpid[j]], w[j], mask=rgbm[j])
            for j in J:
                sticky = sticky | still_a[j] | still_b[j]
            return sticky

        def pass_cond(st):
            return st[1] > 0

        def pass_body(st):
            sticky = lax.fori_loop(0, GROUPS // GUNROLL, supergroup,
                                   jnp.zeros((16,), jnp.bool_))
            return (0, jnp.max(sticky.astype(jnp.int32)))

        lax.while_loop(pass_cond, pass_body, (0, jnp.int32(1)))

    issue(0, 0)
    issue(1, 1)

    def pair_body(cp, _):
        for s in range(2):
            ci = cp * 2 + s
            wait(s, ci)
            process(s, ci)

            @pl.when(ci + 2 < NCHUNK)
            def _():
                issue(s, ci + 2)
        return 0

    lax.fori_loop(0, NCHUNK // 2, pair_body, 0)

    pltpu.sync_copy(keybuf, keyout.at[pl.ds(base_px, SLOT_PIX)])
    pltpu.sync_copy(rgbbuf, rgbout.at[pl.ds(base_px, SLOT_PIX)])


def _zbuffer(uvid, key, rgbw):
    mesh = plsc.VectorSubcoreMesh(core_axis_name="c", subcore_axis_name="s")
    fn = pl.kernel(
        _zbuffer_body,
        out_type=(jax.ShapeDtypeStruct((NB * HW,), jnp.float32),
                  jax.ShapeDtypeStruct((NB * HW,), jnp.int32)),
        mesh=mesh,
        scratch_types=[
            pltpu.VMEM((SLOT_PIX,), jnp.float32),
            pltpu.VMEM((SLOT_PIX,), jnp.int32),
            pltpu.VMEM((SLOT_PIX,), jnp.int32),
            pltpu.VMEM((CHUNK,), jnp.int32),
            pltpu.VMEM((CHUNK,), jnp.float32),
            pltpu.VMEM((CHUNK,), jnp.int32),
            pltpu.VMEM((CHUNK,), jnp.int32),
            pltpu.VMEM((CHUNK,), jnp.float32),
            pltpu.VMEM((CHUNK,), jnp.int32),
            pltpu.SemaphoreType.DMA,
            pltpu.SemaphoreType.DMA,
        ],
        compiler_params=pltpu.CompilerParams(needs_layout_passes=False),
    )
    return fn(uvid, key, rgbw)


def _shift01(x):
    return jnp.concatenate([x[:1], x[:-1]], axis=0)


def _shift10(x):
    return jnp.concatenate([x[:, :1], x[:, :-1]], axis=1)


def _shift11(x):
    inner = jnp.concatenate([x[1:, :1], x[:-1, :-1]], axis=1)
    return jnp.concatenate([x[:1], inner], axis=0)


def _epi_body(key_ref, rgb_ref, img_ref, dep_ref, mask_ref):
    key = key_ref[0]
    w = rgb_ref[0]
    has = key < 1.0e7
    dep = jnp.where(has, key, 1.0e8)
    r = jnp.where(has, ((w >> 21) & 2047).astype(jnp.float32) * (1.0 / 2047.0), 0.0)
    g = jnp.where(has, ((w >> 10) & 2047).astype(jnp.float32) * (1.0 / 2047.0), 0.0)
    b = jnp.where(has, (w & 1023).astype(jnp.float32) * (1.0 / 1023.0), 0.0)

    q = W // 4
    d0 = dep[:, q:3 * q]
    d1 = _shift01(d0)
    d2 = _shift10(d0)
    d3 = _shift11(d0)
    m1 = d1 < d0
    bd = jnp.where(m1, d1, d0)
    m2 = d2 < bd
    bd = jnp.where(m2, d2, bd)
    m3 = d3 < bd
    bd = jnp.where(m3, d3, bd)

    def merge_chan(c):
        c0 = c[:, q:3 * q]
        bc = jnp.where(m1, _shift01(c0), c0)
        bc = jnp.where(m2, _shift10(c0), bc)
        bc = jnp.where(m3, _shift11(c0), bc)
        return jnp.concatenate([c[:, :q], bc, c[:, 3 * q:]], axis=1)

    img_ref[0, 0] = merge_chan(r)
    img_ref[0, 1] = merge_chan(g)
    img_ref[0, 2] = merge_chan(b)
    dep_out = jnp.concatenate([dep[:, :q], bd, dep[:, 3 * q:]], axis=1)
    dep_ref[0, 0] = dep_out
    mask_ref[0, 0] = (dep_out > 1000000.0).astype(jnp.float32)


def _epilogue(keyimg, rgbimg):
    spec2 = pl.BlockSpec((1, H, W), lambda i: (i, 0, 0))
    return pl.pallas_call(
        _epi_body,
        grid=(NB,),
        in_specs=[spec2, spec2],
        out_specs=[pl.BlockSpec((1, 3, H, W), lambda i: (i, 0, 0, 0)),
                   pl.BlockSpec((1, 1, H, W), lambda i: (i, 0, 0, 0)),
                   pl.BlockSpec((1, 1, H, W), lambda i: (i, 0, 0, 0))],
        out_shape=[jax.ShapeDtypeStruct((NB, 3, H, W), jnp.float32),
                   jax.ShapeDtypeStruct((NB, 1, H, W), jnp.float32),
                   jax.ShapeDtypeStruct((NB, 1, H, W), jnp.float32)],
    )(keyimg, rgbimg)


def kernel(img, depth, T):
    u, v, d = _project_points(depth, T)
    u = u.reshape(NB, H, W)
    v = v.reshape(NB, H, W)
    d3 = d.reshape(NB, H, W)
    uvid, rgbw = _prep(u, v, d3, img[:, 0], img[:, 1], img[:, 2])
    keyout, rgbout = _zbuffer(uvid.reshape(NB * HW),
                              d.reshape(NB * HW),
                              rgbw.reshape(NB * HW))
    nv_img, nv_depth, nv_mask = _epilogue(keyout.reshape(NB, H, W),
                                          rgbout.reshape(NB, H, W))
    return nv_img, nv_depth, nv_mask
